# baseline (speedup 1.0000x reference)
import jax
import jax.numpy as jnp
from jax import lax
from jax.experimental import pallas as pl
from jax.experimental.pallas import tpu as pltpu
from jax.experimental.pallas import tpu_sc as plsc

B = 1024
E = 1024
KD = 256
M = 8192
PL = 8
K = 8
KH = K // 2

NW = 32
NB = 4
BB = B // NB
RPW = BB // NW


def _topk_body(x_ref, w_ref, keys_ref, idx_ref, wts_ref):
    x = x_ref[...]
    Wm = w_ref[...]
    proj = lax.dot_general(x, Wm, (((1,), (1,)), ((), ())),
                           preferred_element_type=jnp.float32)
    pn = jnp.sqrt(jnp.sum(proj * proj, axis=1, keepdims=True))
    proj = proj / jnp.maximum(pn, 1e-12)
    keys = keys_ref[...]
    kn = jnp.sqrt(jnp.sum(keys * keys, axis=1, keepdims=True))
    keys = keys / jnp.maximum(kn, 1e-12)
    sim = lax.dot_general(proj, keys, (((1,), (1,)), ((), ())),
                          preferred_element_type=jnp.float32)

    col = lax.broadcasted_iota(jnp.int32, sim.shape, 1)
    scores, idxs = [], []
    for _ in range(K):
        m = jnp.max(sim, axis=1, keepdims=True)
        am = jnp.min(jnp.where(sim == m, col, jnp.int32(M)), axis=1,
                     keepdims=True)
        scores.append(m)
        idxs.append(am)
        sim = jnp.where(col == am, jnp.float32(-jnp.inf), sim)
    s = jnp.concatenate(scores, axis=1)
    ii = jnp.concatenate(idxs, axis=1)
    w = jnp.exp(s - s[:, 0:1])
    w = w / jnp.sum(w, axis=1, keepdims=True)
    z = jnp.zeros_like(ii[:, :4])
    idx_ref[...] = jnp.concatenate([ii[:, :4], z, ii[:, 4:], z], axis=1)
    wts_ref[...] = jnp.pad(w, ((0, 0), (0, 8)))


def _topk_call(x_block, W, prompt_keys):
    return pl.pallas_call(
        _topk_body,
        grid=(1,),
        in_specs=[
            pl.BlockSpec((BB, E), lambda i: (0, 0)),
            pl.BlockSpec((KD, E), lambda i: (0, 0)),
            pl.BlockSpec((M, KD), lambda i: (0, 0)),
        ],
        out_specs=[
            pl.BlockSpec((BB, 16), lambda i: (0, 0)),
            pl.BlockSpec((BB, 16), lambda i: (0, 0)),
        ],
        out_shape=[
            jax.ShapeDtypeStruct((BB, 16), jnp.int32),
            jax.ShapeDtypeStruct((BB, 16), jnp.float32),
        ],
    )(x_block, W, prompt_keys)


def _combine_body(pm_hbm, idx_hbm, wts_hbm, out_hbm,
                  idx_v, wts_v, qa, qb, o0, o1, ga, gb, so0, so1):
    wid = lax.axis_index("s") * 2 + lax.axis_index("c")
    rowbase = wid * RPW
    pltpu.sync_copy(idx_hbm.at[pl.ds(rowbase, RPW)], idx_v)
    pltpu.sync_copy(wts_hbm.at[pl.ds(rowbase, RPW)], wts_v)

    pltpu.async_copy(out_hbm.at[rowbase], o0, so0)
    pltpu.async_copy(out_hbm.at[rowbase], o1, so1)
    pltpu.async_copy(pm_hbm.at[idx_v.at[0, pl.ds(0, KH)]], qa, ga)
    pltpu.async_copy(pm_hbm.at[idx_v.at[0, pl.ds(8, KH)]], qb, gb)

    def pair(t, carry):
        r0 = 2 * t
        r1 = r0 + 1
        rn = jnp.minimum(r0 + 2, RPW - 1)
        for r, rnext, o, so in ((r0, r1, o0, so0), (r1, rn, o1, so1)):
            wv = wts_v[r]
            wb = [wv[jnp.full((16,), k, jnp.int32)] for k in range(K)]
            pltpu.make_async_copy(out_hbm.at[rowbase], o, so).wait()

            pltpu.make_async_copy(
                pm_hbm.at[idx_v.at[r, pl.ds(0, KH)]], qa, ga).wait()

            def chunk_a(cj, c, wb=wb):
                for pp in range(PL):
                    acc = qa[0, pp, pl.ds(cj * 16, 16)] * wb[0]
                    for k in range(1, KH):
                        acc = acc + qa[k, pp, pl.ds(cj * 16, 16)] * wb[k]
                    o[pp, pl.ds(cj * 16, 16)] = acc
                return c

            lax.fori_loop(0, E // 16, chunk_a, 0)
            pltpu.async_copy(
                pm_hbm.at[idx_v.at[rnext, pl.ds(0, KH)]], qa, ga)

            pltpu.make_async_copy(
                pm_hbm.at[idx_v.at[r, pl.ds(8, KH)]], qb, gb).wait()

            def chunk_b(cj, c, wb=wb):
                for pp in range(PL):
                    acc = o[pp, pl.ds(cj * 16, 16)]
                    for k in range(KH):
                        acc = acc + qb[k, pp, pl.ds(cj * 16, 16)] * wb[KH + k]
                    o[pp, pl.ds(cj * 16, 16)] = acc
                return c

            lax.fori_loop(0, E // 16, chunk_b, 0)
            pltpu.async_copy(
                pm_hbm.at[idx_v.at[rnext, pl.ds(8, KH)]], qb, gb)

            pltpu.async_copy(o, out_hbm.at[rowbase + r], so)
        return carry

    lax.fori_loop(0, RPW // 2, pair, 0)
    pltpu.make_async_copy(pm_hbm.at[idx_v.at[0, pl.ds(0, KH)]], qa, ga).wait()
    pltpu.make_async_copy(pm_hbm.at[idx_v.at[0, pl.ds(8, KH)]], qb, gb).wait()
    pltpu.make_async_copy(out_hbm.at[rowbase], o0, so0).wait()
    pltpu.make_async_copy(out_hbm.at[rowbase], o1, so1).wait()


def _combine_call(pm, idx, wts):
    mesh = plsc.VectorSubcoreMesh(core_axis_name="c", subcore_axis_name="s")
    f = pl.kernel(
        _combine_body,
        out_type=jax.ShapeDtypeStruct((BB, PL, E), jnp.float32),
        mesh=mesh,
        scratch_types=[
            pltpu.VMEM((RPW, 16), jnp.int32),
            pltpu.VMEM((RPW, 16), jnp.float32),
            pltpu.VMEM((KH, PL, E), jnp.float32),
            pltpu.VMEM((KH, PL, E), jnp.float32),
            pltpu.VMEM((PL, E), jnp.float32),
            pltpu.VMEM((PL, E), jnp.float32),
            pltpu.SemaphoreType.DMA,
            pltpu.SemaphoreType.DMA,
            pltpu.SemaphoreType.DMA,
            pltpu.SemaphoreType.DMA,
        ],
    )
    return f(pm, idx, wts)


def kernel(x_query, W, prompt_memory, prompt_keys):
    out = jnp.zeros((B, PL, E), jnp.float32)
    for bk in range(NB):
        xb = lax.slice_in_dim(x_query, bk * BB, (bk + 1) * BB, axis=0)
        idx16, wts = _topk_call(xb, W, prompt_keys)
        ob = _combine_call(prompt_memory, idx16, wts)
        out = lax.dynamic_update_slice(out, ob, (bk * BB, 0, 0))
    return out


# speedup vs baseline: 1.1864x; 1.1864x over previous
import jax
import jax.numpy as jnp
from jax import lax
from jax.experimental import pallas as pl
from jax.experimental.pallas import tpu as pltpu
from jax.experimental.pallas import tpu_sc as plsc

B = 1024
E = 1024
KD = 256
M = 8192
PL = 8
K = 8
KH = K // 2

NW = 32
NB = 2
BB = B // NB
RPW = BB // NW


def _topk_body(x_ref, w_ref, keys_ref, idx_ref, wts_ref):
    x = x_ref[...]
    Wm = w_ref[...]
    proj = lax.dot_general(x, Wm, (((1,), (1,)), ((), ())),
                           preferred_element_type=jnp.float32)
    pn = jnp.sqrt(jnp.sum(proj * proj, axis=1, keepdims=True))
    proj = proj / jnp.maximum(pn, 1e-12)
    keys = keys_ref[...]
    kn = jnp.sqrt(jnp.sum(keys * keys, axis=1, keepdims=True))
    keys = keys / jnp.maximum(kn, 1e-12)
    sim = lax.dot_general(proj, keys, (((1,), (1,)), ((), ())),
                          preferred_element_type=jnp.float32)

    col = lax.broadcasted_iota(jnp.int32, sim.shape, 1)
    scores, idxs = [], []
    for _ in range(K):
        m = jnp.max(sim, axis=1, keepdims=True)
        am = jnp.min(jnp.where(sim == m, col, jnp.int32(M)), axis=1,
                     keepdims=True)
        scores.append(m)
        idxs.append(am)
        sim = jnp.where(col == am, jnp.float32(-jnp.inf), sim)
    s = jnp.concatenate(scores, axis=1)
    ii = jnp.concatenate(idxs, axis=1)
    w = jnp.exp(s - s[:, 0:1])
    w = w / jnp.sum(w, axis=1, keepdims=True)
    z = jnp.zeros_like(ii[:, :4])
    idx_ref[...] = jnp.concatenate([ii[:, :4], z, ii[:, 4:], z], axis=1)
    wts_ref[...] = jnp.pad(w, ((0, 0), (0, 8)))


def _topk_call(x_block, W, prompt_keys):
    return pl.pallas_call(
        _topk_body,
        grid=(1,),
        in_specs=[
            pl.BlockSpec((BB, E), lambda i: (0, 0)),
            pl.BlockSpec((KD, E), lambda i: (0, 0)),
            pl.BlockSpec((M, KD), lambda i: (0, 0)),
        ],
        out_specs=[
            pl.BlockSpec((BB, 16), lambda i: (0, 0)),
            pl.BlockSpec((BB, 16), lambda i: (0, 0)),
        ],
        out_shape=[
            jax.ShapeDtypeStruct((BB, 16), jnp.int32),
            jax.ShapeDtypeStruct((BB, 16), jnp.float32),
        ],
    )(x_block, W, prompt_keys)


def _combine_body(pm_hbm, idx_hbm, wts_hbm, out_hbm,
                  idx_v, wts_v, qa, qb, o0, o1, ga, gb, so0, so1):
    wid = lax.axis_index("s") * 2 + lax.axis_index("c")
    rowbase = wid * RPW
    pltpu.sync_copy(idx_hbm.at[pl.ds(rowbase, RPW)], idx_v)
    pltpu.sync_copy(wts_hbm.at[pl.ds(rowbase, RPW)], wts_v)

    pltpu.async_copy(out_hbm.at[rowbase], o0, so0)
    pltpu.async_copy(out_hbm.at[rowbase], o1, so1)
    pltpu.async_copy(pm_hbm.at[idx_v.at[0, pl.ds(0, KH)]], qa, ga)
    pltpu.async_copy(pm_hbm.at[idx_v.at[0, pl.ds(8, KH)]], qb, gb)

    def pair(t, carry):
        r0 = 2 * t
        r1 = r0 + 1
        rn = jnp.minimum(r0 + 2, RPW - 1)
        for r, rnext, o, so in ((r0, r1, o0, so0), (r1, rn, o1, so1)):
            wv = wts_v[r]
            wb = [wv[jnp.full((16,), k, jnp.int32)] for k in range(K)]
            pltpu.make_async_copy(out_hbm.at[rowbase], o, so).wait()

            pltpu.make_async_copy(
                pm_hbm.at[idx_v.at[r, pl.ds(0, KH)]], qa, ga).wait()

            def chunk_a(cj, c, wb=wb):
                for pp in range(PL):
                    acc = qa[0, pp, pl.ds(cj * 16, 16)] * wb[0]
                    for k in range(1, KH):
                        acc = acc + qa[k, pp, pl.ds(cj * 16, 16)] * wb[k]
                    o[pp, pl.ds(cj * 16, 16)] = acc
                return c

            lax.fori_loop(0, E // 16, chunk_a, 0)
            pltpu.async_copy(
                pm_hbm.at[idx_v.at[rnext, pl.ds(0, KH)]], qa, ga)

            pltpu.make_async_copy(
                pm_hbm.at[idx_v.at[r, pl.ds(8, KH)]], qb, gb).wait()

            def chunk_b(cj, c, wb=wb):
                for pp in range(PL):
                    acc = o[pp, pl.ds(cj * 16, 16)]
                    for k in range(KH):
                        acc = acc + qb[k, pp, pl.ds(cj * 16, 16)] * wb[KH + k]
                    o[pp, pl.ds(cj * 16, 16)] = acc
                return c

            lax.fori_loop(0, E // 16, chunk_b, 0)
            pltpu.async_copy(
                pm_hbm.at[idx_v.at[rnext, pl.ds(8, KH)]], qb, gb)

            pltpu.async_copy(o, out_hbm.at[rowbase + r], so)
        return carry

    lax.fori_loop(0, RPW // 2, pair, 0)
    pltpu.make_async_copy(pm_hbm.at[idx_v.at[0, pl.ds(0, KH)]], qa, ga).wait()
    pltpu.make_async_copy(pm_hbm.at[idx_v.at[0, pl.ds(8, KH)]], qb, gb).wait()
    pltpu.make_async_copy(out_hbm.at[rowbase], o0, so0).wait()
    pltpu.make_async_copy(out_hbm.at[rowbase], o1, so1).wait()


def _combine_call(pm, idx, wts):
    mesh = plsc.VectorSubcoreMesh(core_axis_name="c", subcore_axis_name="s")
    f = pl.kernel(
        _combine_body,
        out_type=jax.ShapeDtypeStruct((BB, PL, E), jnp.float32),
        mesh=mesh,
        scratch_types=[
            pltpu.VMEM((RPW, 16), jnp.int32),
            pltpu.VMEM((RPW, 16), jnp.float32),
            pltpu.VMEM((KH, PL, E), jnp.float32),
            pltpu.VMEM((KH, PL, E), jnp.float32),
            pltpu.VMEM((PL, E), jnp.float32),
            pltpu.VMEM((PL, E), jnp.float32),
            pltpu.SemaphoreType.DMA,
            pltpu.SemaphoreType.DMA,
            pltpu.SemaphoreType.DMA,
            pltpu.SemaphoreType.DMA,
        ],
    )
    return f(pm, idx, wts)


def kernel(x_query, W, prompt_memory, prompt_keys):
    out = jnp.zeros((B, PL, E), jnp.float32)
    for bk in range(NB):
        xb = lax.slice_in_dim(x_query, bk * BB, (bk + 1) * BB, axis=0)
        idx16, wts = _topk_call(xb, W, prompt_keys)
        ob = _combine_call(prompt_memory, idx16, wts)
        out = lax.dynamic_update_slice(out, ob, (bk * BB, 0, 0))
    return out
